# Initial kernel scaffold; baseline (speedup 1.0000x reference)
import jax
import jax.numpy as jnp
from jax.experimental import pallas as pl


def kernel(x, edge_index, batch, W1, b1, W2, b2, Wf1, bf1, Wf2, bf2):
    raise NotImplementedError("write your pallas kernel here")


# speedup vs baseline: 25.6047x; 25.6047x over previous
import functools

import jax
import jax.numpy as jnp
from jax import lax
from jax.experimental import pallas as pl
from jax.experimental.pallas import tpu as pltpu
from jax.experimental.pallas import tpu_sc as plsc

NC = 2
NS = 16
NW = NC * NS

_B = 64


def _sc_mesh():
    return plsc.VectorSubcoreMesh(
        core_axis_name="c", subcore_axis_name="s", num_cores=NC,
        num_subcores=NS)


def _make_deg_kernel(N, E):
    ept = E // NW
    K = 2000
    nchunk = ept // K

    @functools.partial(
        pl.kernel,
        out_type=jax.ShapeDtypeStruct((NC, N, 1), jnp.float32),
        mesh=_sc_mesh(),
        scratch_types=[
            pltpu.VMEM((K,), jnp.int32),
            pltpu.VMEM((K, 1), jnp.float32),
            pltpu.VMEM_SHARED((N, 1), jnp.float32),
        ],
        compiler_params=pltpu.CompilerParams(use_tc_tiling_on_sc=False),
    )
    def deg_kernel(dst_hbm, ones_hbm, zeros_hbm, out_hbm, idx_v, ones_v, acc_sh):
        c = lax.axis_index("c")
        s = lax.axis_index("s")

        pltpu.sync_copy(ones_hbm, ones_v)

        @pl.when(s == 0)
        def _():
            pltpu.sync_copy(zeros_hbm, acc_sh)
        plsc.subcore_barrier()

        base = (c * NS + s) * ept

        def chunk(i, carry):
            off = base + i * K
            pltpu.sync_copy(dst_hbm.at[pl.ds(off, K)], idx_v)
            pltpu.sync_copy(ones_v, acc_sh.at[idx_v], add=True)
            return carry
        lax.fori_loop(0, nchunk, chunk, 0)

        plsc.subcore_barrier()

        @pl.when(s == 0)
        def _():
            pltpu.sync_copy(acc_sh, out_hbm.at[c])

    return deg_kernel


def _make_agg_kernel(N, E, H):
    ept = E // NW
    K = 1000
    nchunk = ept // K
    nps = N // NS

    @functools.partial(
        pl.kernel,
        out_type=jax.ShapeDtypeStruct((NC, N, H), jnp.float32),
        mesh=_sc_mesh(),
        scratch_types=[
            pltpu.VMEM((K,), jnp.int32),
            pltpu.VMEM((K,), jnp.int32),
            pltpu.VMEM((K, H), jnp.float32),
            pltpu.VMEM_SHARED((N, H), jnp.float32),
            pltpu.SemaphoreType.DMA,
        ],
        compiler_params=pltpu.CompilerParams(use_tc_tiling_on_sc=False),
    )
    def agg_kernel(ht_hbm, src_hbm, dst_hbm, zeros_hbm, out_hbm,
                   src_v, dst_v, rows_v, acc_sh, sem):
        c = lax.axis_index("c")
        s = lax.axis_index("s")

        pltpu.sync_copy(zeros_hbm.at[pl.ds(s * nps, nps)],
                        acc_sh.at[pl.ds(s * nps, nps)])
        plsc.subcore_barrier()

        base = (c * NS + s) * ept

        def chunk(i, carry):
            off = base + i * K
            pltpu.sync_copy(src_hbm.at[pl.ds(off, K)], src_v)
            pltpu.sync_copy(dst_hbm.at[pl.ds(off, K)], dst_v)
            pltpu.async_copy(ht_hbm.at[src_v], rows_v, sem).wait()
            pltpu.sync_copy(rows_v, acc_sh.at[dst_v], add=True)
            return carry
        lax.fori_loop(0, nchunk, chunk, 0)

        plsc.subcore_barrier()
        pltpu.sync_copy(acc_sh.at[pl.ds(s * nps, nps)],
                        out_hbm.at[c, pl.ds(s * nps, nps)])

    return agg_kernel


def _pre_body(x_ref, w1_ref, degp_ref, h_ref, ht_ref, dinv_ref):
    h = jnp.dot(x_ref[...], w1_ref[...], preferred_element_type=jnp.float32)
    deg = degp_ref[0] + degp_ref[1] + 1.0
    dinv = lax.rsqrt(deg)
    h_ref[...] = h
    ht_ref[...] = h * dinv
    dinv_ref[...] = dinv


def _mid_body(p_ref, h_ref, dinv_ref, b_ref, w2_ref, h2_ref, ht2_ref):
    dinv = dinv_ref[...]
    agg = (p_ref[0] + p_ref[1]) * dinv + h_ref[...] * (dinv * dinv)
    h1p = jnp.maximum(agg + b_ref[...], 0.0)
    h2 = jnp.dot(h1p, w2_ref[...], preferred_element_type=jnp.float32)
    h2_ref[...] = h2
    ht2_ref[...] = h2 * dinv


def _relu_agg_body(p_ref, h_ref, dinv_ref, b_ref, out_ref):
    dinv = dinv_ref[...]
    agg = (p_ref[0] + p_ref[1]) * dinv + h_ref[...] * (dinv * dinv)
    out_ref[...] = jnp.maximum(agg + b_ref[...], 0.0)


def _segmax_body(h_ref, batch_ref, out_ref):
    b = pl.program_id(0)
    mask = batch_ref[...] == b
    masked = jnp.where(mask, h_ref[...], -jnp.inf)
    out_ref[...] = jnp.max(masked, axis=0, keepdims=True)[None]


def _head_body(h_ref, batch_row_ref, xmax_ref, wf1_ref, bf1_ref,
               wf2_ref, bf2_ref, out_ref):
    n = h_ref.shape[0]
    iota = lax.broadcasted_iota(jnp.int32, (_B, n), 0)
    onehot = (batch_row_ref[...] == iota).astype(jnp.float32)
    seg_sum = jnp.dot(onehot, h_ref[...],
                      preferred_element_type=jnp.float32)
    cnt = jnp.sum(onehot, axis=1, keepdims=True)
    mean = seg_sum / jnp.maximum(cnt, 1.0)
    g = jnp.concatenate([mean, xmax_ref[...]], axis=1)
    gf = jnp.maximum(
        jnp.dot(g, wf1_ref[...], preferred_element_type=jnp.float32)
        + bf1_ref[...], 0.0)
    z = jnp.dot(gf, wf2_ref[...], preferred_element_type=jnp.float32) \
        + bf2_ref[...]
    out_ref[...] = 1.0 / (1.0 + jnp.exp(-z))


def kernel(x, edge_index, batch, W1, b1, W2, b2, Wf1, bf1, Wf2, bf2):
    N, F_in = x.shape
    H = W1.shape[1]
    E = edge_index.shape[1]

    src = edge_index[0]
    dst = edge_index[1]
    zeros_col = jnp.zeros((N, 1), jnp.float32)
    zeros_nh = jnp.zeros((N, H), jnp.float32)

    deg_fn = _make_deg_kernel(N, E)
    agg_fn = _make_agg_kernel(N, E, H)

    ones_chunk = jnp.ones((2000, 1), jnp.float32)
    deg_p = deg_fn(dst, ones_chunk, zeros_col)

    h1, ht1, dinv = pl.pallas_call(
        _pre_body,
        out_shape=(
            jax.ShapeDtypeStruct((N, H), jnp.float32),
            jax.ShapeDtypeStruct((N, H), jnp.float32),
            jax.ShapeDtypeStruct((N, 1), jnp.float32),
        ),
    )(x, W1, deg_p)

    p1 = agg_fn(ht1, src, dst, zeros_nh)

    h2, ht2 = pl.pallas_call(
        _mid_body,
        out_shape=(
            jax.ShapeDtypeStruct((N, H), jnp.float32),
            jax.ShapeDtypeStruct((N, H), jnp.float32),
        ),
    )(p1, h1, dinv, b1.reshape(1, H), W2)

    p2 = agg_fn(ht2, src, dst, zeros_nh)

    h2p = pl.pallas_call(
        _relu_agg_body,
        out_shape=jax.ShapeDtypeStruct((N, H), jnp.float32),
    )(p2, h2, dinv, b2.reshape(1, H))

    batch_col = batch.reshape(N, 1)
    xmax = pl.pallas_call(
        _segmax_body,
        grid=(_B,),
        in_specs=[
            pl.BlockSpec((N, H), lambda b: (0, 0)),
            pl.BlockSpec((N, 1), lambda b: (0, 0)),
        ],
        out_specs=pl.BlockSpec((1, 1, H), lambda b: (b, 0, 0)),
        out_shape=jax.ShapeDtypeStruct((_B, 1, H), jnp.float32),
    )(h2p, batch_col)
    xmax = xmax.reshape(_B, H)

    out = pl.pallas_call(
        _head_body,
        out_shape=jax.ShapeDtypeStruct((_B, 1), jnp.float32),
    )(h2p, batch.reshape(1, N), xmax, Wf1, bf1.reshape(1, H),
      Wf2, bf2.reshape(1, 1))

    return out


# speedup vs baseline: 33.1231x; 1.2936x over previous
import functools

import jax
import jax.numpy as jnp
from jax import lax
from jax.experimental import pallas as pl
from jax.experimental.pallas import tpu as pltpu
from jax.experimental.pallas import tpu_sc as plsc

NC = 2
NS = 16
NW = NC * NS

_B = 64


def _sc_mesh():
    return plsc.VectorSubcoreMesh(
        core_axis_name="c", subcore_axis_name="s", num_cores=NC,
        num_subcores=NS)


def _make_deg_kernel(N, E):
    ept = E // NW
    K = 2000
    nchunk = ept // K
    NIB = 4

    @functools.partial(
        pl.kernel,
        out_type=jax.ShapeDtypeStruct((NC, N, 1), jnp.float32),
        mesh=_sc_mesh(),
        scratch_types=[
            pltpu.VMEM((NIB, K), jnp.int32),
            pltpu.VMEM((K, 1), jnp.float32),
            pltpu.VMEM_SHARED((N, 1), jnp.float32),
        ]
        + [pltpu.SemaphoreType.DMA] * (NIB + 2 + 1),
        compiler_params=pltpu.CompilerParams(use_tc_tiling_on_sc=False),
    )
    def deg_kernel(dst_hbm, ones_hbm, zeros_hbm, out_hbm, idx_v, ones_v,
                   acc_sh, *sems):
        isems = sems[:NIB]
        ssems = sems[NIB:NIB + 2]
        zsem = sems[NIB + 2]
        c = lax.axis_index("c")
        s = lax.axis_index("s")
        base = (c * NS + s) * ept

        def idx_start(i):
            return pltpu.async_copy(
                dst_hbm.at[pl.ds(base + i * K, K)], idx_v.at[i % NIB],
                isems[i % NIB])

        def scat_start(i):
            return pltpu.async_copy(
                ones_v, acc_sh.at[idx_v.at[i % NIB]], ssems[i % 2], add=True)

        @pl.when(s == 0)
        def _():
            pltpu.async_copy(zeros_hbm, acc_sh, zsem)

        idx_d = {i: idx_start(i) for i in range(min(3, nchunk))}
        pltpu.sync_copy(ones_hbm, ones_v)

        @pl.when(s == 0)
        def _():
            pltpu.make_async_copy(zeros_hbm, acc_sh, zsem).wait()
        plsc.subcore_barrier()

        scat_d = {}
        for i in range(nchunk):
            idx_d[i].wait()
            if i >= 1:
                scat_d[i - 1].wait()
            scat_d[i] = scat_start(i)
            if i + 3 < nchunk:
                idx_d[i + 3] = idx_start(i + 3)
        scat_d[nchunk - 1].wait()

        plsc.subcore_barrier()

        @pl.when(s == 0)
        def _():
            pltpu.sync_copy(acc_sh, out_hbm.at[c])

    return deg_kernel


def _make_agg_kernel(N, E, H):
    ept = E // NW
    K = 400
    nchunk = ept // K
    nps = N // NS
    NIB = 4
    NRB = 3

    @functools.partial(
        pl.kernel,
        out_type=jax.ShapeDtypeStruct((NC, N, H), jnp.float32),
        mesh=_sc_mesh(),
        scratch_types=[
            pltpu.VMEM((NIB, K), jnp.int32),
            pltpu.VMEM((NIB, K), jnp.int32),
            pltpu.VMEM((NRB, K, H), jnp.float32),
            pltpu.VMEM_SHARED((N, H), jnp.float32),
        ]
        + [pltpu.SemaphoreType.DMA] * (NIB + 2 * NRB + 1),
        compiler_params=pltpu.CompilerParams(use_tc_tiling_on_sc=False),
    )
    def agg_kernel(ht_hbm, src_hbm, dst_hbm, zeros_hbm, out_hbm,
                   src_v, dst_v, rows_v, acc_sh, *sems):
        isems = sems[:NIB]
        gsems = sems[NIB:NIB + NRB]
        ssems = sems[NIB + NRB:NIB + 2 * NRB]
        zsem = sems[NIB + 2 * NRB]
        c = lax.axis_index("c")
        s = lax.axis_index("s")
        base = (c * NS + s) * ept

        def idx_start(i):
            off = base + i * K
            j = i % NIB
            return (
                pltpu.async_copy(src_hbm.at[pl.ds(off, K)], src_v.at[j],
                                 isems[j]),
                pltpu.async_copy(dst_hbm.at[pl.ds(off, K)], dst_v.at[j],
                                 isems[j]),
            )

        def gather_start(i):
            return pltpu.async_copy(ht_hbm.at[src_v.at[i % NIB]],
                                    rows_v.at[i % NRB], gsems[i % NRB])

        def scat_start(i):
            return pltpu.async_copy(rows_v.at[i % NRB],
                                    acc_sh.at[dst_v.at[i % NIB]],
                                    ssems[i % NRB], add=True)

        zd = pltpu.async_copy(zeros_hbm.at[pl.ds(s * nps, nps)],
                              acc_sh.at[pl.ds(s * nps, nps)], zsem)
        idx_d = {i: idx_start(i) for i in range(min(3, nchunk))}
        gat_d = {}
        for i in range(min(2, nchunk)):
            for d in idx_d[i]:
                d.wait()
            gat_d[i] = gather_start(i)
        zd.wait()
        plsc.subcore_barrier()

        scat_d = {}
        for i in range(nchunk):
            gat_d[i].wait()
            if i >= 1:
                scat_d[i - 1].wait()
            scat_d[i] = scat_start(i)
            if i + 2 < nchunk:
                for d in idx_d[i + 2]:
                    d.wait()
                gat_d[i + 2] = gather_start(i + 2)
            if i + 3 < nchunk:
                idx_d[i + 3] = idx_start(i + 3)
        scat_d[nchunk - 1].wait()

        plsc.subcore_barrier()
        pltpu.sync_copy(acc_sh.at[pl.ds(s * nps, nps)],
                        out_hbm.at[c, pl.ds(s * nps, nps)])

    return agg_kernel


def _pre_body(x_ref, w1_ref, degp_ref, h_ref, ht_ref, dinv_ref):
    h = jnp.dot(x_ref[...], w1_ref[...], preferred_element_type=jnp.float32)
    deg = degp_ref[0] + degp_ref[1] + 1.0
    dinv = lax.rsqrt(deg)
    h_ref[...] = h
    ht_ref[...] = h * dinv
    dinv_ref[...] = dinv


def _mid_body(p_ref, h_ref, dinv_ref, b_ref, w2_ref, h2_ref, ht2_ref):
    dinv = dinv_ref[...]
    agg = (p_ref[0] + p_ref[1]) * dinv + h_ref[...] * (dinv * dinv)
    h1p = jnp.maximum(agg + b_ref[...], 0.0)
    h2 = jnp.dot(h1p, w2_ref[...], preferred_element_type=jnp.float32)
    h2_ref[...] = h2
    ht2_ref[...] = h2 * dinv


def _make_post_body(nblocks, H):
    def post_body(p_ref, h_ref, dinv_ref, b2_ref, batch_col_ref,
                  batch_row_ref, wf1_ref, bf1_ref, wf2_ref, bf2_ref,
                  out_ref, xmax_acc, sum_acc, cnt_acc):
        k = pl.program_id(0)
        bs = h_ref.shape[0]

        dinv = dinv_ref[...]
        agg = (p_ref[0] + p_ref[1]) * dinv + h_ref[...] * (dinv * dinv)
        h2p = jnp.maximum(agg + b2_ref[...], 0.0)
        bc = batch_col_ref[...]

        @pl.when(k == 0)
        def _():
            xmax_acc[...] = jnp.full((_B, H), -jnp.inf, jnp.float32)
            sum_acc[...] = jnp.zeros((_B, H), jnp.float32)
            cnt_acc[...] = jnp.zeros((_B, 1), jnp.float32)

        iota = lax.broadcasted_iota(jnp.int32, (_B, bs), 0)
        onehot = (batch_row_ref[0] == iota).astype(jnp.float32)
        sum_acc[...] += jnp.dot(onehot, h2p,
                                preferred_element_type=jnp.float32)
        cnt_acc[...] += jnp.sum(onehot, axis=1, keepdims=True)

        for b in range(_B):
            m = jnp.max(jnp.where(bc == b, h2p, -jnp.inf), axis=0,
                        keepdims=True)
            xmax_acc[b:b + 1, :] = jnp.maximum(xmax_acc[b:b + 1, :], m)

        @pl.when(k == nblocks - 1)
        def _():
            mean = sum_acc[...] / jnp.maximum(cnt_acc[...], 1.0)
            g = jnp.concatenate([mean, xmax_acc[...]], axis=1)
            gf = jnp.maximum(
                jnp.dot(g, wf1_ref[...], preferred_element_type=jnp.float32)
                + bf1_ref[...], 0.0)
            z = jnp.dot(gf, wf2_ref[...],
                        preferred_element_type=jnp.float32) + bf2_ref[...]
            out_ref[...] = 1.0 / (1.0 + jnp.exp(-z))

    return post_body


def kernel(x, edge_index, batch, W1, b1, W2, b2, Wf1, bf1, Wf2, bf2):
    N, F_in = x.shape
    H = W1.shape[1]
    E = edge_index.shape[1]

    src = edge_index[0]
    dst = edge_index[1]
    zeros_col = jnp.zeros((N, 1), jnp.float32)
    zeros_nh = jnp.zeros((N, H), jnp.float32)

    deg_fn = _make_deg_kernel(N, E)
    agg_fn = _make_agg_kernel(N, E, H)

    ones_chunk = jnp.ones((2000, 1), jnp.float32)
    deg_p = deg_fn(dst, ones_chunk, zeros_col)

    h1, ht1, dinv = pl.pallas_call(
        _pre_body,
        out_shape=(
            jax.ShapeDtypeStruct((N, H), jnp.float32),
            jax.ShapeDtypeStruct((N, H), jnp.float32),
            jax.ShapeDtypeStruct((N, 1), jnp.float32),
        ),
    )(x, W1, deg_p)

    p1 = agg_fn(ht1, src, dst, zeros_nh)

    h2, ht2 = pl.pallas_call(
        _mid_body,
        out_shape=(
            jax.ShapeDtypeStruct((N, H), jnp.float32),
            jax.ShapeDtypeStruct((N, H), jnp.float32),
        ),
    )(p1, h1, dinv, b1.reshape(1, H), W2)

    p2 = agg_fn(ht2, src, dst, zeros_nh)

    nblocks = 10
    bs = N // nblocks
    out = pl.pallas_call(
        _make_post_body(nblocks, H),
        grid=(nblocks,),
        in_specs=[
            pl.BlockSpec((2, bs, H), lambda k: (0, k, 0)),
            pl.BlockSpec((bs, H), lambda k: (k, 0)),
            pl.BlockSpec((bs, 1), lambda k: (k, 0)),
            pl.BlockSpec((1, H), lambda k: (0, 0)),
            pl.BlockSpec((bs, 1), lambda k: (k, 0)),
            pl.BlockSpec((1, 1, bs), lambda k: (k, 0, 0)),
            pl.BlockSpec((2 * H, H), lambda k: (0, 0)),
            pl.BlockSpec((1, H), lambda k: (0, 0)),
            pl.BlockSpec((H, 1), lambda k: (0, 0)),
            pl.BlockSpec((1, 1), lambda k: (0, 0)),
        ],
        out_specs=pl.BlockSpec((_B, 1), lambda k: (0, 0)),
        out_shape=jax.ShapeDtypeStruct((_B, 1), jnp.float32),
        scratch_shapes=[
            pltpu.VMEM((_B, H), jnp.float32),
            pltpu.VMEM((_B, H), jnp.float32),
            pltpu.VMEM((_B, 1), jnp.float32),
        ],
    )(p2, h2, dinv, b2.reshape(1, H), batch.reshape(N, 1),
      batch.reshape(nblocks, 1, bs), Wf1, bf1.reshape(1, H), Wf2,
      bf2.reshape(1, 1))

    return out
